# baseline (speedup 1.0000x reference)
import functools

import jax
import jax.numpy as jnp
from jax import lax
from jax.experimental import pallas as pl
from jax.experimental.pallas import tpu as pltpu
from jax.experimental.pallas import tpu_sc as plsc

NC = 2
NS = 16
NW = NC * NS

NBUF = 4
K = 2


@functools.lru_cache(maxsize=None)
def _make(B, W, n_chunks, chunk):
    mesh = plsc.VectorSubcoreMesh(
        core_axis_name="c", subcore_axis_name="s",
        num_cores=NC, num_subcores=NS)
    b_per_w = n_chunks * chunk
    assert n_chunks % NBUF == 0

    @functools.partial(
        pl.kernel,
        out_type=jax.ShapeDtypeStruct((B, W), jnp.float32),
        mesh=mesh,
        scratch_types=[
            pltpu.VMEM((n_chunks, chunk), jnp.int32),
            pltpu.VMEM((NBUF, chunk, W), jnp.float32),
            pltpu.SemaphoreType.DMA((NBUF,)),
            pltpu.SemaphoreType.DMA((NBUF,)),
        ],
        compiler_params=pltpu.CompilerParams(use_tc_tiling_on_sc=False),
    )
    def k(idx_hbm, table_hbm, out_hbm, idx_v, bufs, gsem, osem):
        wid = lax.axis_index("s") * NC + lax.axis_index("c")
        base = wid * b_per_w
        pltpu.sync_copy(idx_hbm.at[wid], idx_v)

        def gather(c, b):
            return pltpu.make_async_copy(
                table_hbm.at[idx_v.at[c]], bufs.at[b], gsem.at[b])

        def write(c, b):
            return pltpu.make_async_copy(
                bufs.at[b, :, pl.ds(0, 64)],
                out_hbm.at[pl.ds(base + c * chunk, chunk), pl.ds(0, 64)],
                osem.at[b])

        for c in range(K):
            gather(c, c % NBUF).start()

        def step(g, carry):
            for b in range(NBUF):
                c = g * NBUF + b
                a = c + K
                ba = (b + K) % NBUF
                w = a - NBUF

                @pl.when(a < n_chunks)
                def _():
                    @pl.when(w >= 0)
                    def _():
                        write(w, ba).wait()
                    gather(a, ba).start()

                gather(c, b).wait()
                write(c, b).start()
            return carry

        lax.fori_loop(0, n_chunks // NBUF, step, 0)

        for c in range(n_chunks - NBUF, n_chunks):
            write(c, c % NBUF).wait()

    return k


@functools.lru_cache(maxsize=None)
def _make_transpose(V, W):
    mesh = plsc.VectorSubcoreMesh(
        core_axis_name="c", subcore_axis_name="s",
        num_cores=NC, num_subcores=NS)
    D = 64
    CB = 128
    nfull = V // CB
    tail = V - nfull * CB
    nblk = nfull + 1
    bpw = (nblk + NW - 1) // NW

    @functools.partial(
        pl.kernel,
        out_type=jax.ShapeDtypeStruct((V, W), jnp.float32),
        mesh=mesh,
        scratch_types=[
            pltpu.VMEM((D, CB), jnp.float32),
            pltpu.VMEM((CB, W), jnp.float32),
            pltpu.VMEM((tail, D), jnp.float32),
        ],
        compiler_params=pltpu.CompilerParams(
            use_tc_tiling_on_sc=True, needs_layout_passes=False),
    )
    def t(tt_hbm, tail_hbm, out_hbm, in_v, out_v, tl_v):
        wid = lax.axis_index("s") * NC + lax.axis_index("c")
        rows = [lax.iota(jnp.int32, 16) + r0 for r0 in range(0, D, 16)]

        def trans(src, ncols):
            for c in range(ncols):
                col = jnp.full((16,), c, jnp.int32)
                for ri, r0 in enumerate(range(0, D, 16)):
                    out_v[c, pl.ds(r0, 16)] = plsc.load_gather(
                        src, [rows[ri], col])

        def body(b, carry):
            j = wid * bpw + b

            @pl.when(j < nfull)
            def _():
                c0 = pl.multiple_of(j * CB, CB)
                pltpu.sync_copy(tt_hbm.at[:, pl.ds(c0, CB)], in_v)
                trans(in_v, CB)
                pltpu.sync_copy(out_v, out_hbm.at[pl.ds(c0, CB), :])
            return carry

        lax.fori_loop(0, bpw, body, 0)

        @pl.when(wid == 0)
        def _():
            pltpu.sync_copy(tail_hbm, tl_v)
            for r0 in range(0, tail, 16):
                for c0 in range(0, D, 16):
                    for i in range(16):
                        out_v[r0 + i, pl.ds(c0, 16)] = tl_v[r0 + i,
                                                            pl.ds(c0, 16)]
            pltpu.sync_copy(out_v.at[pl.ds(0, tail), :],
                            out_hbm.at[pl.ds(nfull * CB, tail), :])

    return t


def kernel(inputs, table):
    S0, S1 = inputs.shape
    B = S0 * S1
    V, D = table.shape
    W = 128
    chunk = 128
    n_chunks = B // (NW * chunk)
    idx = inputs.reshape(NW, n_chunks, chunk).astype(jnp.int32)
    table_p = _make_transpose(V, W)(table.T, table[V - 64:])
    out_p = _make(B, W, n_chunks, chunk)(idx, table_p)
    return out_p.reshape(S0, S1, W)[:, :, :D]


# speedup vs baseline: 2.5005x; 2.5005x over previous
import functools

import jax
import jax.numpy as jnp
from jax import lax
from jax.experimental import pallas as pl
from jax.experimental.pallas import tpu as pltpu
from jax.experimental.pallas import tpu_sc as plsc

NC = 2
NS = 16
NW = NC * NS

NBUF = 4
K = 3


@functools.lru_cache(maxsize=None)
def _make(B, W, n_chunks, chunk):
    mesh = plsc.VectorSubcoreMesh(
        core_axis_name="c", subcore_axis_name="s",
        num_cores=NC, num_subcores=NS)
    b_per_w = n_chunks * chunk
    assert n_chunks % NBUF == 0

    @functools.partial(
        pl.kernel,
        out_type=jax.ShapeDtypeStruct((B, W), jnp.float32),
        mesh=mesh,
        scratch_types=[
            pltpu.VMEM((n_chunks, chunk), jnp.int32),
            pltpu.VMEM((NBUF, chunk, W), jnp.float32),
            pltpu.SemaphoreType.DMA((NBUF,)),
            pltpu.SemaphoreType.DMA((NBUF,)),
        ],
        compiler_params=pltpu.CompilerParams(use_tc_tiling_on_sc=False),
    )
    def k(idx_hbm, table_hbm, out_hbm, idx_v, bufs, gsem, osem):
        wid = lax.axis_index("s") * NC + lax.axis_index("c")
        base = wid * b_per_w
        pltpu.sync_copy(idx_hbm.at[wid], idx_v)

        def gather(c, b):
            return pltpu.make_async_copy(
                table_hbm.at[idx_v.at[c]], bufs.at[b], gsem.at[b])

        def write(c, b):
            return pltpu.make_async_copy(
                bufs.at[b, :, pl.ds(0, 64)],
                out_hbm.at[pl.ds(base + c * chunk, chunk), pl.ds(0, 64)],
                osem.at[b])

        for c in range(K):
            gather(c, c % NBUF).start()

        def step(g, carry):
            for b in range(NBUF):
                c = g * NBUF + b
                a = c + K
                ba = (b + K) % NBUF
                w = a - NBUF

                @pl.when(a < n_chunks)
                def _():
                    @pl.when(w >= 0)
                    def _():
                        write(w, ba).wait()
                    gather(a, ba).start()

                gather(c, b).wait()
                write(c, b).start()
            return carry

        lax.fori_loop(0, n_chunks // NBUF, step, 0)

        for c in range(n_chunks - NBUF, n_chunks):
            write(c, c % NBUF).wait()

    return k


def kernel(inputs, table):
    S0, S1 = inputs.shape
    B = S0 * S1
    V, D = table.shape
    W = 128
    chunk = 128
    n_chunks = B // (NW * chunk)
    idx = inputs.reshape(NW, n_chunks, chunk).astype(jnp.int32)
    table_p = jnp.pad(table, ((0, 0), (0, W - D)))
    out_p = _make(B, W, n_chunks, chunk)(idx, table_p)
    return out_p.reshape(S0, S1, W)[:, :, :D]
